# baseline (speedup 1.0000x reference)
import jax
import jax.numpy as jnp
from jax import lax
from jax.experimental import pallas as pl
from jax.experimental.pallas import tpu as pltpu
from jax.experimental.pallas import tpu_sc as plsc

N = 10000
E = 160000
D = 256
DH = 128
NS = 16
CB = 176
NCHUNK = 58
EPT = CB * NCHUNK
EPAD = NS * EPT
RPT = 624
VL = 16


def _spmm_kernel(h_hbm, src_hbm, dst_hbm, ew_hbm, out_hbm,
                 src_a, src_b, dst_a, dst_b, ew_a, ew_b, sdst_a, sdst_b,
                 rows_a, rows_b, acc,
                 gsem_a, gsem_b, ssem_a, ssem_b, io_a, io_b):
    c = lax.axis_index("c")
    s = lax.axis_index("s")
    bufs = ((src_a, dst_a, ew_a, sdst_a, rows_a, gsem_a, ssem_a, io_a),
            (src_b, dst_b, ew_b, sdst_b, rows_b, gsem_b, ssem_b, io_b))

    def _issue_idx(k, par):
        src_v, dst_v, ew_v, _, _, _, _, io = bufs[par]
        base = s * EPT + k * CB
        pltpu.async_copy(src_hbm.at[pl.ds(c * EPAD + base, CB)], src_v, io)
        pltpu.async_copy(dst_hbm.at[pl.ds(base, CB)], dst_v, io)
        pltpu.async_copy(ew_hbm.at[pl.ds(base, CB)], ew_v, io)

    def _wait_idx(par):
        src_v, dst_v, ew_v, _, _, _, _, io = bufs[par]
        pltpu.make_async_copy(src_hbm.at[pl.ds(0, CB)], src_v, io).wait()
        pltpu.make_async_copy(dst_hbm.at[pl.ds(0, CB)], dst_v, io).wait()
        pltpu.make_async_copy(ew_hbm.at[pl.ds(0, CB)], ew_v, io).wait()

    def _zrow(i, _):
        for j in range(DH // VL):
            rows_a[i, pl.ds(j * VL, VL)] = jnp.zeros((VL,), jnp.float32)
        return 0
    lax.fori_loop(0, CB, _zrow, 0)
    _issue_idx(0, 0)
    _issue_idx(1, 1)
    r0 = s * RPT
    pltpu.sync_copy(rows_a, acc.at[pl.ds(r0, CB)])
    pltpu.sync_copy(rows_a, acc.at[pl.ds(r0 + CB, CB)])
    pltpu.sync_copy(rows_a, acc.at[pl.ds(r0 + 2 * CB, CB)])
    pltpu.sync_copy(rows_a.at[pl.ds(0, RPT - 3 * CB)],
                    acc.at[pl.ds(r0 + 3 * CB, RPT - 3 * CB)])

    @pl.when(s == NS - 1)
    def _zero_tail():
        pltpu.sync_copy(rows_a.at[pl.ds(0, N - NS * RPT)],
                        acc.at[pl.ds(NS * RPT, N - NS * RPT)])
    plsc.subcore_barrier()

    _wait_idx(0)
    pltpu.async_copy(h_hbm.at[src_a], rows_a, gsem_a)

    def _step(k, par):
        src_v, dst_v, ew_v, sdst_v, rows_v, gsem, ssem, _ = bufs[par]
        n_src, n_dst, n_ew, n_sdst, n_rows, n_gsem, n_ssem, _ = bufs[1 - par]

        @pl.when(k > 0)
        def _wait_prev_scatter():
            pltpu.make_async_copy(n_rows, acc.at[n_sdst], n_ssem).wait()

        @pl.when(k < NCHUNK - 1)
        def _issue_next_gather():
            _wait_idx(1 - par)
            pltpu.async_copy(h_hbm.at[n_src], n_rows, n_gsem)

        pltpu.make_async_copy(h_hbm.at[src_v], rows_v, gsem).wait()

        def _wgroup(g, _):
            wv = ew_v[pl.ds(g * VL, VL)]
            for l in range(VL):
                w = wv[l]
                i = g * VL + l
                for j in range(DH // VL):
                    rows_v[i, pl.ds(j * VL, VL)] = (
                        rows_v[i, pl.ds(j * VL, VL)] * w)
            return 0
        lax.fori_loop(0, CB // VL, _wgroup, 0)
        for j in range(CB // VL):
            sdst_v[pl.ds(j * VL, VL)] = dst_v[pl.ds(j * VL, VL)]
        pltpu.async_copy(rows_v, acc.at[sdst_v], ssem, add=True)

        @pl.when(k < NCHUNK - 2)
        def _prefetch_idx():
            _issue_idx(k + 2, par)

    def _pair(t, _):
        _step(2 * t, 0)
        _step(2 * t + 1, 1)
        return 0
    lax.fori_loop(0, NCHUNK // 2, _pair, 0)
    pltpu.make_async_copy(rows_b, acc.at[sdst_b], ssem_b).wait()
    plsc.subcore_barrier()

    pltpu.sync_copy(acc.at[pl.ds(r0, RPT)],
                    out_hbm.at[pl.ds(c * N + r0, RPT)])

    @pl.when(s == NS - 1)
    def _out_tail():
        pltpu.sync_copy(acc.at[pl.ds(NS * RPT, N - NS * RPT)],
                        out_hbm.at[pl.ds(c * N + NS * RPT, N - NS * RPT)])


_spmm = pl.kernel(
    _spmm_kernel,
    out_type=jax.ShapeDtypeStruct((2 * N, DH), jnp.float32),
    mesh=plsc.VectorSubcoreMesh(core_axis_name="c", subcore_axis_name="s"),
    scratch_types=[
        pltpu.VMEM((CB,), jnp.int32),
        pltpu.VMEM((CB,), jnp.int32),
        pltpu.VMEM((CB,), jnp.int32),
        pltpu.VMEM((CB,), jnp.int32),
        pltpu.VMEM((CB,), jnp.float32),
        pltpu.VMEM((CB,), jnp.float32),
        pltpu.VMEM((CB,), jnp.int32),
        pltpu.VMEM((CB,), jnp.int32),
        pltpu.VMEM((CB, DH), jnp.float32),
        pltpu.VMEM((CB, DH), jnp.float32),
        pltpu.VMEM_SHARED((N, DH), jnp.float32),
        pltpu.SemaphoreType.DMA,
        pltpu.SemaphoreType.DMA,
        pltpu.SemaphoreType.DMA,
        pltpu.SemaphoreType.DMA,
        pltpu.SemaphoreType.DMA,
        pltpu.SemaphoreType.DMA,
    ],
)


def _fused_body(agg_ref, h_ref, wrel_ref, wroot_ref, b_ref, out_ref):
    a = jnp.concatenate([agg_ref[0], agg_ref[1]], axis=1)
    h = jnp.concatenate([h_ref[0], h_ref[1]], axis=1)
    out = (jnp.dot(a + h, wrel_ref[...], preferred_element_type=jnp.float32)
           + jnp.dot(h, wroot_ref[...], preferred_element_type=jnp.float32)
           + b_ref[...])
    out_ref[0] = out[:, :DH]
    out_ref[1] = out[:, DH:]


_RB = 1000


def _fused(agg_cat, h_cat, w_rel, w_root, b2d):
    return pl.pallas_call(
        _fused_body,
        grid=(N // _RB,),
        in_specs=[
            pl.BlockSpec((2, _RB, DH), lambda i: (0, i, 0)),
            pl.BlockSpec((2, _RB, DH), lambda i: (0, i, 0)),
            pl.BlockSpec((D, D), lambda i: (0, 0)),
            pl.BlockSpec((D, D), lambda i: (0, 0)),
            pl.BlockSpec((1, D), lambda i: (0, 0)),
        ],
        out_specs=pl.BlockSpec((2, _RB, DH), lambda i: (0, i, 0)),
        out_shape=jax.ShapeDtypeStruct((2, N, DH), jnp.float32),
    )(agg_cat, h_cat, w_rel, w_root, b2d)


def kernel(x, edge_index, edge_weight, W1_rel, W1_root, b1, W2_rel, W2_root, b2):
    src = edge_index[0].astype(jnp.int32)
    dst = edge_index[1].astype(jnp.int32)
    perm = jnp.argsort(src)
    src = src[perm]
    dst = dst[perm]
    edge_weight = edge_weight[perm]
    npad = EPAD - E
    zpad = jnp.zeros((npad,), jnp.int32)
    src_p = jnp.concatenate([src, zpad])
    dst_p = jnp.concatenate([dst, zpad])
    ew_p = jnp.concatenate([edge_weight, jnp.zeros((npad,), jnp.float32)])
    src2 = jnp.concatenate([src_p, src_p + N])

    h = x.reshape(N, 2, DH).transpose(1, 0, 2)
    layers = [(W1_rel, W1_root, b1.reshape(1, D))] + \
             [(W2_rel, W2_root, b2.reshape(1, D))] * 4
    for w_rel, w_root, b2d in layers:
        agg = _spmm(h.reshape(2 * N, DH), src2, dst_p, ew_p)
        h = _fused(agg.reshape(2, N, DH), h, w_rel, w_root, b2d)
    return h.transpose(1, 0, 2).reshape(N, D)


# speedup vs baseline: 1.8771x; 1.8771x over previous
import jax
import jax.numpy as jnp
from jax import lax
from jax.experimental import pallas as pl
from jax.experimental.pallas import tpu as pltpu
from jax.experimental.pallas import tpu_sc as plsc

N = 10000
E = 160000
D = 256
DH = 128
NS = 16
CB = 112
NCHUNK = 90
EPT = CB * NCHUNK
EPAD = NS * EPT
RPT = 624
VL = 16


def _spmm_kernel(h_hbm, src_hbm, dst_hbm, ew_hbm, out_hbm,
                 src_a, src_b, src_c, dst_a, dst_b, dst_c,
                 ew_a, ew_b, ew_c, sdst_a, sdst_b, sdst_c,
                 rows_a, rows_b, rows_c, acc,
                 gsem_a, gsem_b, gsem_c, ssem_a, ssem_b, ssem_c,
                 io_a, io_b, io_c):
    c = lax.axis_index("c")
    s = lax.axis_index("s")
    bufs = ((src_a, dst_a, ew_a, sdst_a, rows_a, gsem_a, ssem_a, io_a),
            (src_b, dst_b, ew_b, sdst_b, rows_b, gsem_b, ssem_b, io_b),
            (src_c, dst_c, ew_c, sdst_c, rows_c, gsem_c, ssem_c, io_c))

    def _issue_idx(k, par):
        src_v, dst_v, ew_v, _, _, _, _, io = bufs[par]
        base = s * EPT + k * CB
        pltpu.async_copy(src_hbm.at[pl.ds(c * EPAD + base, CB)], src_v, io)
        pltpu.async_copy(dst_hbm.at[pl.ds(base, CB)], dst_v, io)
        pltpu.async_copy(ew_hbm.at[pl.ds(base, CB)], ew_v, io)

    def _wait_idx(par):
        src_v, dst_v, ew_v, _, _, _, _, io = bufs[par]
        pltpu.make_async_copy(src_hbm.at[pl.ds(0, CB)], src_v, io).wait()
        pltpu.make_async_copy(dst_hbm.at[pl.ds(0, CB)], dst_v, io).wait()
        pltpu.make_async_copy(ew_hbm.at[pl.ds(0, CB)], ew_v, io).wait()

    def _issue_gather(par):
        src_v, _, _, _, rows_v, gsem, _, _ = bufs[par]
        pltpu.async_copy(h_hbm.at[src_v], rows_v, gsem)

    def _zrow(i, _):
        for j in range(DH // VL):
            rows_a[i, pl.ds(j * VL, VL)] = jnp.zeros((VL,), jnp.float32)
        return 0
    lax.fori_loop(0, CB, _zrow, 0)
    _issue_idx(0, 0)
    _issue_idx(1, 1)
    _issue_idx(2, 2)
    r0 = s * RPT
    for i in range(RPT // CB):
        pltpu.sync_copy(rows_a, acc.at[pl.ds(r0 + i * CB, CB)])
    _ZREM = RPT - (RPT // CB) * CB
    pltpu.sync_copy(rows_a.at[pl.ds(0, _ZREM)],
                    acc.at[pl.ds(r0 + RPT - _ZREM, _ZREM)])

    @pl.when(s == NS - 1)
    def _zero_tail():
        pltpu.sync_copy(rows_a.at[pl.ds(0, N - NS * RPT)],
                        acc.at[pl.ds(NS * RPT, N - NS * RPT)])
    plsc.subcore_barrier()

    _wait_idx(0)
    _issue_gather(0)
    _wait_idx(1)
    _issue_gather(1)

    def _step(k, par):
        src_v, dst_v, ew_v, sdst_v, rows_v, gsem, ssem, _ = bufs[par]
        p_src, p_dst, p_ew, p_sdst, p_rows, p_gsem, p_ssem, _ = \
            bufs[(par + 2) % 3]

        @pl.when(k > 0)
        def _wait_prev_scatter():
            pltpu.make_async_copy(p_rows, acc.at[p_sdst], p_ssem).wait()

        @pl.when(k < NCHUNK - 2)
        def _issue_next_gather():
            _wait_idx((par + 2) % 3)
            _issue_gather((par + 2) % 3)

        pltpu.make_async_copy(h_hbm.at[src_v], rows_v, gsem).wait()

        def _wgroup(g, _):
            wv = ew_v[pl.ds(g * VL, VL)]
            for l in range(VL):
                w = wv[l]
                i = g * VL + l
                for j in range(DH // VL):
                    rows_v[i, pl.ds(j * VL, VL)] = (
                        rows_v[i, pl.ds(j * VL, VL)] * w)
            return 0
        lax.fori_loop(0, CB // VL, _wgroup, 0)
        for j in range(CB // VL):
            sdst_v[pl.ds(j * VL, VL)] = dst_v[pl.ds(j * VL, VL)]
        pltpu.async_copy(rows_v, acc.at[sdst_v], ssem, add=True)

        @pl.when(k < NCHUNK - 3)
        def _prefetch_idx():
            _issue_idx(k + 3, par)

    def _trip(t, _):
        _step(3 * t, 0)
        _step(3 * t + 1, 1)
        _step(3 * t + 2, 2)
        return 0
    lax.fori_loop(0, NCHUNK // 3, _trip, 0)
    pltpu.make_async_copy(rows_c, acc.at[sdst_c], ssem_c).wait()
    plsc.subcore_barrier()

    pltpu.sync_copy(acc.at[pl.ds(r0, RPT)],
                    out_hbm.at[pl.ds(c * N + r0, RPT)])

    @pl.when(s == NS - 1)
    def _out_tail():
        pltpu.sync_copy(acc.at[pl.ds(NS * RPT, N - NS * RPT)],
                        out_hbm.at[pl.ds(c * N + NS * RPT, N - NS * RPT)])


_spmm = pl.kernel(
    _spmm_kernel,
    out_type=jax.ShapeDtypeStruct((2 * N, DH), jnp.float32),
    mesh=plsc.VectorSubcoreMesh(core_axis_name="c", subcore_axis_name="s"),
    scratch_types=(
        [pltpu.VMEM((CB,), jnp.int32)] * 3
        + [pltpu.VMEM((CB,), jnp.int32)] * 3
        + [pltpu.VMEM((CB,), jnp.float32)] * 3
        + [pltpu.VMEM((CB,), jnp.int32)] * 3
        + [pltpu.VMEM((CB, DH), jnp.float32)] * 3
        + [pltpu.VMEM_SHARED((N, DH), jnp.float32)]
        + [pltpu.SemaphoreType.DMA] * 9
    ),
)


def _fused_body(agg_ref, h_ref, wrel_ref, wroot_ref, b_ref, out_ref):
    a = jnp.concatenate([agg_ref[0], agg_ref[1]], axis=1)
    h = jnp.concatenate([h_ref[0], h_ref[1]], axis=1)
    out = (jnp.dot(a + h, wrel_ref[...], preferred_element_type=jnp.float32)
           + jnp.dot(h, wroot_ref[...], preferred_element_type=jnp.float32)
           + b_ref[...])
    out_ref[0] = out[:, :DH]
    out_ref[1] = out[:, DH:]


_RB = 1000


def _fused(agg_cat, h_cat, w_rel, w_root, b2d):
    return pl.pallas_call(
        _fused_body,
        grid=(N // _RB,),
        in_specs=[
            pl.BlockSpec((2, _RB, DH), lambda i: (0, i, 0)),
            pl.BlockSpec((2, _RB, DH), lambda i: (0, i, 0)),
            pl.BlockSpec((D, D), lambda i: (0, 0)),
            pl.BlockSpec((D, D), lambda i: (0, 0)),
            pl.BlockSpec((1, D), lambda i: (0, 0)),
        ],
        out_specs=pl.BlockSpec((2, _RB, DH), lambda i: (0, i, 0)),
        out_shape=jax.ShapeDtypeStruct((2, N, DH), jnp.float32),
    )(agg_cat, h_cat, w_rel, w_root, b2d)


def kernel(x, edge_index, edge_weight, W1_rel, W1_root, b1, W2_rel, W2_root, b2):
    src = edge_index[0].astype(jnp.int32)
    dst = edge_index[1].astype(jnp.int32)
    npad = EPAD - E
    zpad = jnp.zeros((npad,), jnp.int32)
    src_p = jnp.concatenate([src, zpad])
    dst_p = jnp.concatenate([dst, zpad])
    ew_p = jnp.concatenate([edge_weight, jnp.zeros((npad,), jnp.float32)])
    src2 = jnp.concatenate([src_p, src_p + N])

    h = x.reshape(N, 2, DH).transpose(1, 0, 2)
    layers = [(W1_rel, W1_root, b1.reshape(1, D))] + \
             [(W2_rel, W2_root, b2.reshape(1, D))] * 4
    for w_rel, w_root, b2d in layers:
        agg = _spmm(h.reshape(2 * N, DH), src2, dst_p, ew_p)
        h = _fused(agg.reshape(2, N, DH), h, w_rel, w_root, b2d)
    return h.transpose(1, 0, 2).reshape(N, D)
